# baseline (speedup 1.0000x reference)
import functools

import jax
import jax.numpy as jnp
from jax import lax
from jax.experimental import pallas as pl
from jax.experimental.pallas import tpu as pltpu
from jax.experimental.pallas import tpu_sc as plsc

N = 10000
H = 128
E = 320000
CHUNK = 128
NCHUNKS = E // CHUNK
NC, NS, NW = 2, 16, 32
ROW_STRIDE = 624
ROW_SPAN = 640
F32 = jnp.float32


BF16 = jnp.bfloat16


def _b0_body(h_ref, wa_ref, wb_ref, p_ref, q_ref):
    hb = h_ref[...]
    p_ref[...] = jnp.dot(hb, wa_ref[...],
                         preferred_element_type=F32).astype(BF16)
    q_ref[...] = jnp.dot(hb, wb_ref[...],
                         preferred_element_type=F32).astype(BF16)


def _edge_body(ps_ref, qd_ref, geo_ref, w1c_ref, b1_ref, w2_ref, b2_ref,
               wc1_ref, bc1_ref, wc2_ref, m_ref, cpos_ref, cneg_ref):
    g = geo_ref[...]
    dist = jnp.sqrt(g[:, 3:4])
    pre = (ps_ref[...].astype(F32) + qd_ref[...].astype(F32)
           + dist * w1c_ref[...] + b1_ref[...])
    t1 = jax.nn.silu(pre)
    mm = jax.nn.silu(jnp.dot(t1, w2_ref[...], preferred_element_type=F32)
                     + b2_ref[...])
    m_ref[...] = mm
    t2 = jax.nn.silu(jnp.dot(mm, wc1_ref[...], preferred_element_type=F32)
                     + bc1_ref[...])
    cw = jnp.dot(t2, wc2_ref[...], preferred_element_type=F32)
    s = cw / (dist + 1e-8)
    cvec = g[:, 0:3] * s
    ones = jnp.ones_like(cw)
    zeros = jnp.zeros(cvec.shape[:1] + (12,), F32)
    cpos_ref[...] = jnp.concatenate([cvec, ones, zeros], axis=1)
    cneg_ref[...] = jnp.concatenate([-cvec, ones, zeros], axis=1)


def _node_body(h_ref, x_ref, agg_ref, cacc_ref, wn1a_ref, wn1b_ref, bn1_ref,
               wn2_ref, bn2_ref, ho_ref, xo_ref):
    a3 = agg_ref[...]
    a = a3[0] + a3[1]
    hb = h_ref[...]
    t = jax.nn.silu(jnp.dot(hb, wn1a_ref[...], preferred_element_type=F32)
                    + jnp.dot(a, wn1b_ref[...], preferred_element_type=F32)
                    + bn1_ref[...])
    ho_ref[...] = hb + jnp.dot(t, wn2_ref[...], preferred_element_type=F32) \
        + bn2_ref[...]
    c3 = cacc_ref[...]
    c = c3[0] + c3[1]
    cnt = jnp.maximum(c[:, 3:4], 1.0)
    xo_ref[...] = x_ref[...] + c[:, 0:3] / cnt


_MESH = plsc.VectorSubcoreMesh(core_axis_name="c", subcore_axis_name="s")


@functools.partial(
    pl.kernel,
    out_type=(jax.ShapeDtypeStruct((E, H // 2), jnp.int32),
              jax.ShapeDtypeStruct((E, H // 2), jnp.int32),
              jax.ShapeDtypeStruct((E, 4), F32)),
    mesh=_MESH,
    scratch_types=[
        pltpu.VMEM((N,), F32), pltpu.VMEM((N,), F32), pltpu.VMEM((N,), F32),
        pltpu.VMEM((CHUNK,), jnp.int32), pltpu.VMEM((CHUNK,), jnp.int32),
        pltpu.VMEM((CHUNK,), jnp.int32), pltpu.VMEM((CHUNK,), jnp.int32),
        pltpu.VMEM((CHUNK, H // 2), jnp.int32), pltpu.VMEM((CHUNK, H // 2), jnp.int32),
        pltpu.VMEM((CHUNK, H // 2), jnp.int32), pltpu.VMEM((CHUNK, H // 2), jnp.int32),
        pltpu.VMEM((CHUNK, 4), F32),
        pltpu.SemaphoreType.DMA, pltpu.SemaphoreType.DMA,
        pltpu.SemaphoreType.DMA, pltpu.SemaphoreType.DMA,
    ],
    compiler_params=pltpu.CompilerParams(needs_layout_passes=False,
                                         use_tc_tiling_on_sc=False),
)
def _sc_gather(p_hbm, q_hbm, src_hbm, dst_hbm, xx_hbm, xy_hbm, xz_hbm,
               ps_hbm, qd_hbm, geo_hbm,
               xx_v, xy_v, xz_v, sidx0, sidx1, didx0, didx1,
               pbuf0, pbuf1, qbuf0, qbuf1, gbuf,
               sem_p0, sem_p1, sem_q0, sem_q1):
    cc = lax.axis_index("c")
    ss = lax.axis_index("s")
    wid = ss * NC + cc
    pltpu.sync_copy(xx_hbm, xx_v)
    pltpu.sync_copy(xy_hbm, xy_v)
    pltpu.sync_copy(xz_hbm, xz_v)
    iota = lax.iota(jnp.int32, 16)
    sidx = (sidx0, sidx1)
    didx = (didx0, didx1)
    pbuf = (pbuf0, pbuf1)
    qbuf = (qbuf0, qbuf1)
    sem_p = (sem_p0, sem_p1)
    sem_q = (sem_q0, sem_q1)
    niter = (NCHUNKS + NW - 1) // NW

    def fire(slot, n):
        e0 = n * CHUNK
        pltpu.sync_copy(src_hbm.at[pl.ds(e0, CHUNK)], sidx[slot])
        pltpu.sync_copy(dst_hbm.at[pl.ds(e0, CHUNK)], didx[slot])
        pltpu.async_copy(p_hbm.at[sidx[slot]], pbuf[slot], sem_p[slot])
        pltpu.async_copy(q_hbm.at[didx[slot]], qbuf[slot], sem_q[slot])

    @pl.when(wid < NCHUNKS)
    def _prologue():
        fire(0, wid)

    @pl.loop(0, (niter + 1) // 2 * 2, step=2)
    def _outer(k):
        for b in range(2):
            p, np_ = b, 1 - b
            n = wid + (k + b) * NW

            @pl.when(n < NCHUNKS)
            def _(p=p, np_=np_, n=n):
                e0 = n * CHUNK
                n_next = n + NW

                @pl.when(n_next < NCHUNKS)
                def _():
                    fire(np_, n_next)

                for sub in range(CHUNK // 16):
                    sv = sidx[p][pl.ds(sub * 16, 16)]
                    dv = didx[p][pl.ds(sub * 16, 16)]
                    dxv = plsc.load_gather(xx_v, [dv]) - plsc.load_gather(xx_v, [sv])
                    dyv = plsc.load_gather(xy_v, [dv]) - plsc.load_gather(xy_v, [sv])
                    dzv = plsc.load_gather(xz_v, [dv]) - plsc.load_gather(xz_v, [sv])
                    sqv = dxv * dxv + dyv * dyv + dzv * dzv
                    rows = iota + (sub * 16)
                    for comp, val in ((0, dxv), (1, dyv), (2, dzv), (3, sqv)):
                        plsc.store_scatter(
                            gbuf, [rows, jnp.full((16,), comp, jnp.int32)], val)
                pltpu.sync_copy(gbuf, geo_hbm.at[pl.ds(e0, CHUNK)])
                pltpu.make_async_copy(p_hbm.at[sidx[p]], pbuf[p], sem_p[p]).wait()
                pltpu.sync_copy(pbuf[p], ps_hbm.at[pl.ds(e0, CHUNK)])
                pltpu.make_async_copy(q_hbm.at[didx[p]], qbuf[p], sem_q[p]).wait()
                pltpu.sync_copy(qbuf[p], qd_hbm.at[pl.ds(e0, CHUNK)])


_CHUNKS_PER_SC = NCHUNKS // NC
_ZROWS = 16


@functools.partial(
    pl.kernel,
    out_type=jax.ShapeDtypeStruct((NC, N, H), F32),
    mesh=_MESH,
    scratch_types=[
        pltpu.VMEM_SHARED((N, H), F32),
        pltpu.VMEM((_ZROWS, H), F32),
        pltpu.VMEM((CHUNK, H), F32), pltpu.VMEM((CHUNK, H), F32),
        pltpu.VMEM((CHUNK,), jnp.int32), pltpu.VMEM((CHUNK,), jnp.int32),
        pltpu.VMEM((CHUNK,), jnp.int32), pltpu.VMEM((CHUNK,), jnp.int32),
        pltpu.SemaphoreType.DMA, pltpu.SemaphoreType.DMA,
    ],
    compiler_params=pltpu.CompilerParams(needs_layout_passes=False),
)
def _sc_scatter_m(m_hbm, src_hbm, dst_hbm, agg_out,
                  agg_sp, zbuf, mbuf0, mbuf1, sidx0, sidx1, didx0, didx1,
                  sem_m0, sem_m1):
    cc = lax.axis_index("c")
    ss = lax.axis_index("s")
    z16 = jnp.zeros((16,), F32)
    mbuf = (mbuf0, mbuf1)
    sidx = (sidx0, sidx1)
    didx = (didx0, didx1)
    sem_m = (sem_m0, sem_m1)

    @pl.loop(0, _ZROWS)
    def _zero(i):
        for j in range(H // 16):
            zbuf[i, pl.ds(j * 16, 16)] = z16

    @pl.loop(0, ROW_SPAN // _ZROWS)
    def _zfill(j):
        pltpu.sync_copy(zbuf, agg_sp.at[pl.ds(ss * ROW_STRIDE + j * _ZROWS,
                                               _ZROWS)])
    plsc.subcore_barrier()

    def fire(slot, j):
        e0 = (cc * _CHUNKS_PER_SC + j) * CHUNK
        pltpu.sync_copy(src_hbm.at[pl.ds(e0, CHUNK)], sidx[slot])
        pltpu.sync_copy(dst_hbm.at[pl.ds(e0, CHUNK)], didx[slot])
        pltpu.async_copy(m_hbm.at[pl.ds(e0, CHUNK)], mbuf[slot], sem_m[slot])

    @pl.when(ss < _CHUNKS_PER_SC)
    def _prologue():
        fire(0, ss)

    niter = (_CHUNKS_PER_SC + NS - 1) // NS

    @pl.loop(0, (niter + 1) // 2 * 2, step=2)
    def _outer(k):
        for b in range(2):
            p, np_ = b, 1 - b
            j = ss + (k + b) * NS

            @pl.when(j < _CHUNKS_PER_SC)
            def _(p=p, np_=np_, j=j):
                @pl.when(j + NS < _CHUNKS_PER_SC)
                def _():
                    fire(np_, j + NS)
                e0 = (cc * _CHUNKS_PER_SC + j) * CHUNK
                pltpu.make_async_copy(m_hbm.at[pl.ds(e0, CHUNK)],
                                      mbuf[p], sem_m[p]).wait()
                pltpu.sync_copy(mbuf[p], agg_sp.at[didx[p]], add=True)
                pltpu.sync_copy(mbuf[p], agg_sp.at[sidx[p]], add=True)

    plsc.subcore_barrier()
    r0 = ss * ROW_STRIDE
    pltpu.sync_copy(agg_sp.at[pl.ds(r0, ROW_SPAN)],
                    agg_out.at[cc, pl.ds(r0, ROW_SPAN)])


@functools.partial(
    pl.kernel,
    out_type=jax.ShapeDtypeStruct((NC, N, 16), F32),
    mesh=_MESH,
    scratch_types=[
        pltpu.VMEM_SHARED((N, 16), F32),
        pltpu.VMEM((_ZROWS, 16), F32),
        pltpu.VMEM((CHUNK, 16), F32), pltpu.VMEM((CHUNK, 16), F32),
        pltpu.VMEM((CHUNK, 16), F32), pltpu.VMEM((CHUNK, 16), F32),
        pltpu.VMEM((CHUNK,), jnp.int32), pltpu.VMEM((CHUNK,), jnp.int32),
        pltpu.VMEM((CHUNK,), jnp.int32), pltpu.VMEM((CHUNK,), jnp.int32),
        pltpu.SemaphoreType.DMA, pltpu.SemaphoreType.DMA,
        pltpu.SemaphoreType.DMA, pltpu.SemaphoreType.DMA,
    ],
    compiler_params=pltpu.CompilerParams(needs_layout_passes=False,
                                         use_tc_tiling_on_sc=False),
)
def _sc_scatter_c(cpos_hbm, cneg_hbm, src_hbm, dst_hbm, cacc_out,
                  cacc_sp, zbuf2, cbuf0, cbuf1, cbuf20, cbuf21,
                  sidx0, sidx1, didx0, didx1,
                  sem_c0, sem_c1, sem_n0, sem_n1):
    cc = lax.axis_index("c")
    ss = lax.axis_index("s")
    z16 = jnp.zeros((16,), F32)
    cbuf = (cbuf0, cbuf1)
    cbuf2 = (cbuf20, cbuf21)
    sidx = (sidx0, sidx1)
    didx = (didx0, didx1)
    sem_c = (sem_c0, sem_c1)
    sem_n = (sem_n0, sem_n1)

    @pl.loop(0, _ZROWS)
    def _zero(i):
        zbuf2[i, :] = z16

    @pl.loop(0, ROW_SPAN // _ZROWS)
    def _zfill(j):
        pltpu.sync_copy(zbuf2, cacc_sp.at[pl.ds(ss * ROW_STRIDE + j * _ZROWS,
                                                _ZROWS)])
    plsc.subcore_barrier()

    def fire(slot, j):
        e0 = (cc * _CHUNKS_PER_SC + j) * CHUNK
        pltpu.sync_copy(src_hbm.at[pl.ds(e0, CHUNK)], sidx[slot])
        pltpu.sync_copy(dst_hbm.at[pl.ds(e0, CHUNK)], didx[slot])
        pltpu.async_copy(cpos_hbm.at[pl.ds(e0, CHUNK)], cbuf[slot], sem_c[slot])
        pltpu.async_copy(cneg_hbm.at[pl.ds(e0, CHUNK)], cbuf2[slot], sem_n[slot])

    @pl.when(ss < _CHUNKS_PER_SC)
    def _prologue():
        fire(0, ss)

    niter = (_CHUNKS_PER_SC + NS - 1) // NS

    @pl.loop(0, (niter + 1) // 2 * 2, step=2)
    def _outer(k):
        for b in range(2):
            p, np_ = b, 1 - b
            j = ss + (k + b) * NS

            @pl.when(j < _CHUNKS_PER_SC)
            def _(p=p, np_=np_, j=j):
                @pl.when(j + NS < _CHUNKS_PER_SC)
                def _():
                    fire(np_, j + NS)
                e0 = (cc * _CHUNKS_PER_SC + j) * CHUNK
                pltpu.make_async_copy(cpos_hbm.at[pl.ds(e0, CHUNK)],
                                      cbuf[p], sem_c[p]).wait()
                pltpu.sync_copy(cbuf[p], cacc_sp.at[didx[p]], add=True)
                pltpu.make_async_copy(cneg_hbm.at[pl.ds(e0, CHUNK)],
                                      cbuf2[p], sem_n[p]).wait()
                pltpu.sync_copy(cbuf2[p], cacc_sp.at[sidx[p]], add=True)

    plsc.subcore_barrier()
    r0 = ss * ROW_STRIDE
    pltpu.sync_copy(cacc_sp.at[pl.ds(r0, ROW_SPAN)],
                    cacc_out.at[cc, pl.ds(r0, ROW_SPAN)])


_BN = 1000
_BE = 2000


def kernel(h, x, bond_indices, W1, b1, W2, b2, Wn1, bn1, Wn2, bn2,
           Wc1, bc1, Wc2):
    h2 = h[0]
    x2 = x[0]
    src = bond_indices[:, 0].astype(jnp.int32)
    dst = bond_indices[:, 1].astype(jnp.int32)
    xx, xy, xz = x2[:, 0], x2[:, 1], x2[:, 2]
    w1a, w1b, w1c = W1[:H], W1[H:2 * H], W1[2 * H:2 * H + 1]
    b1r, b2r, bc1r = b1[None], b2[None], bc1[None]
    wn1a, wn1b = Wn1[:H], Wn1[H:2 * H]
    bn1r, bn2r = bn1[None], bn2[None]

    p_arr, q_arr = pl.pallas_call(
        _b0_body,
        grid=(N // _BN,),
        in_specs=[pl.BlockSpec((_BN, H), lambda i: (i, 0)),
                  pl.BlockSpec((H, H), lambda i: (0, 0)),
                  pl.BlockSpec((H, H), lambda i: (0, 0))],
        out_specs=[pl.BlockSpec((_BN, H), lambda i: (i, 0)),
                   pl.BlockSpec((_BN, H), lambda i: (i, 0))],
        out_shape=[jax.ShapeDtypeStruct((N, H), jnp.bfloat16),
                   jax.ShapeDtypeStruct((N, H), jnp.bfloat16)],
    )(h2, w1a, w1b)

    p32 = jax.lax.bitcast_convert_type(
        p_arr.reshape(N, H // 2, 2), jnp.int32)
    q32 = jax.lax.bitcast_convert_type(
        q_arr.reshape(N, H // 2, 2), jnp.int32)
    ps32, qd32, geo = _sc_gather(p32, q32, src, dst, xx, xy, xz)
    ps = jax.lax.bitcast_convert_type(ps32, jnp.bfloat16).reshape(E, H)
    qd = jax.lax.bitcast_convert_type(qd32, jnp.bfloat16).reshape(E, H)

    m_arr, cpos, cneg = pl.pallas_call(
        _edge_body,
        grid=(E // _BE,),
        in_specs=[pl.BlockSpec((_BE, H), lambda i: (i, 0)),
                  pl.BlockSpec((_BE, H), lambda i: (i, 0)),
                  pl.BlockSpec((_BE, 4), lambda i: (i, 0)),
                  pl.BlockSpec((1, H), lambda i: (0, 0)),
                  pl.BlockSpec((1, H), lambda i: (0, 0)),
                  pl.BlockSpec((H, H), lambda i: (0, 0)),
                  pl.BlockSpec((1, H), lambda i: (0, 0)),
                  pl.BlockSpec((H, H), lambda i: (0, 0)),
                  pl.BlockSpec((1, H), lambda i: (0, 0)),
                  pl.BlockSpec((H, 1), lambda i: (0, 0))],
        out_specs=[pl.BlockSpec((_BE, H), lambda i: (i, 0)),
                   pl.BlockSpec((_BE, 16), lambda i: (i, 0)),
                   pl.BlockSpec((_BE, 16), lambda i: (i, 0))],
        out_shape=[jax.ShapeDtypeStruct((E, H), F32),
                   jax.ShapeDtypeStruct((E, 16), F32),
                   jax.ShapeDtypeStruct((E, 16), F32)],
    )(ps, qd, geo, w1c, b1r, W2, b2r, Wc1, bc1r, Wc2)

    aggp = _sc_scatter_m(m_arr, src, dst)
    caccp = _sc_scatter_c(cpos, cneg, src, dst)

    h_out, x_out = pl.pallas_call(
        _node_body,
        grid=(N // _BN,),
        in_specs=[pl.BlockSpec((_BN, H), lambda i: (i, 0)),
                  pl.BlockSpec((_BN, 3), lambda i: (i, 0)),
                  pl.BlockSpec((NC, _BN, H), lambda i: (0, i, 0)),
                  pl.BlockSpec((NC, _BN, 16), lambda i: (0, i, 0)),
                  pl.BlockSpec((H, H), lambda i: (0, 0)),
                  pl.BlockSpec((H, H), lambda i: (0, 0)),
                  pl.BlockSpec((1, H), lambda i: (0, 0)),
                  pl.BlockSpec((H, H), lambda i: (0, 0)),
                  pl.BlockSpec((1, H), lambda i: (0, 0))],
        out_specs=[pl.BlockSpec((_BN, H), lambda i: (i, 0)),
                   pl.BlockSpec((_BN, 3), lambda i: (i, 0))],
        out_shape=[jax.ShapeDtypeStruct((N, H), F32),
                   jax.ShapeDtypeStruct((N, 3), F32)],
    )(h2, x2, aggp, caccp, wn1a, wn1b, bn1r, Wn2, bn2r)

    return h_out[None], x_out[None]


# speedup vs baseline: 2.2703x; 2.2703x over previous
import functools

import jax
import jax.numpy as jnp
from jax import lax
from jax.experimental import pallas as pl
from jax.experimental.pallas import tpu as pltpu
from jax.experimental.pallas import tpu_sc as plsc

N = 10000
H = 128
E = 320000
CHUNK = 128
NCHUNKS = E // CHUNK
NC, NS, NW = 2, 16, 32
ROW_STRIDE = 624
ROW_SPAN = 640
F32 = jnp.float32


def _b0_body(h_ref, wa_ref, wb_ref, p_ref, q_ref):
    hb = h_ref[...]
    p_ref[...] = jnp.dot(hb, wa_ref[...], preferred_element_type=F32)
    q_ref[...] = jnp.dot(hb, wb_ref[...], preferred_element_type=F32)


def _edge_body(ps_ref, qd_ref, geo_ref, w1c_ref, b1_ref, w2_ref, b2_ref,
               wc1_ref, bc1_ref, wc2_ref, m_ref, cpos_ref, cneg_ref):
    g = geo_ref[...]
    dist = jnp.sqrt(g[:, 3:4])
    pre = ps_ref[...] + qd_ref[...] + dist * w1c_ref[...] + b1_ref[...]
    t1 = jax.nn.silu(pre)
    mm = jax.nn.silu(jnp.dot(t1, w2_ref[...], preferred_element_type=F32)
                     + b2_ref[...])
    m_ref[...] = mm
    t2 = jax.nn.silu(jnp.dot(mm, wc1_ref[...], preferred_element_type=F32)
                     + bc1_ref[...])
    cw = jnp.dot(t2, wc2_ref[...], preferred_element_type=F32)
    s = cw / (dist + 1e-8)
    cvec = g[:, 0:3] * s
    ones = jnp.ones_like(cw)
    zeros = jnp.zeros(cvec.shape[:1] + (12,), F32)
    cpos_ref[...] = jnp.concatenate([cvec, ones, zeros], axis=1)
    cneg_ref[...] = jnp.concatenate([-cvec, ones, zeros], axis=1)


def _node_body(h_ref, x_ref, agg_ref, cacc_ref, wn1a_ref, wn1b_ref, bn1_ref,
               wn2_ref, bn2_ref, ho_ref, xo_ref):
    a3 = agg_ref[...]
    a = a3[0] + a3[1]
    hb = h_ref[...]
    t = jax.nn.silu(jnp.dot(hb, wn1a_ref[...], preferred_element_type=F32)
                    + jnp.dot(a, wn1b_ref[...], preferred_element_type=F32)
                    + bn1_ref[...])
    ho_ref[...] = hb + jnp.dot(t, wn2_ref[...], preferred_element_type=F32) \
        + bn2_ref[...]
    c3 = cacc_ref[...]
    c = c3[0] + c3[1]
    cnt = jnp.maximum(c[:, 3:4], 1.0)
    xo_ref[...] = x_ref[...] + c[:, 0:3] / cnt


_MESH = plsc.VectorSubcoreMesh(core_axis_name="c", subcore_axis_name="s")


@functools.partial(
    pl.kernel,
    out_type=(jax.ShapeDtypeStruct((E, H), F32),
              jax.ShapeDtypeStruct((E, H), F32),
              jax.ShapeDtypeStruct((E, 4), F32)),
    mesh=_MESH,
    scratch_types=[
        pltpu.VMEM((N,), F32), pltpu.VMEM((N,), F32), pltpu.VMEM((N,), F32),
        pltpu.VMEM((CHUNK,), jnp.int32), pltpu.VMEM((CHUNK,), jnp.int32),
        pltpu.VMEM((CHUNK,), jnp.int32), pltpu.VMEM((CHUNK,), jnp.int32),
        pltpu.VMEM((CHUNK, H), F32), pltpu.VMEM((CHUNK, H), F32),
        pltpu.VMEM((CHUNK, H), F32), pltpu.VMEM((CHUNK, H), F32),
        pltpu.VMEM((CHUNK, 4), F32),
        pltpu.SemaphoreType.DMA, pltpu.SemaphoreType.DMA,
        pltpu.SemaphoreType.DMA, pltpu.SemaphoreType.DMA,
        pltpu.SemaphoreType.DMA, pltpu.SemaphoreType.DMA,
        pltpu.SemaphoreType.DMA, pltpu.SemaphoreType.DMA,
    ],
    compiler_params=pltpu.CompilerParams(needs_layout_passes=False),
)
def _sc_gather(p_hbm, q_hbm, src_hbm, dst_hbm, xx_hbm, xy_hbm, xz_hbm,
               ps_hbm, qd_hbm, geo_hbm,
               xx_v, xy_v, xz_v, sidx0, sidx1, didx0, didx1,
               pbuf0, pbuf1, qbuf0, qbuf1, gbuf,
               sem_p0, sem_p1, sem_q0, sem_q1,
               sem_wp0, sem_wp1, sem_wq0, sem_wq1):
    cc = lax.axis_index("c")
    ss = lax.axis_index("s")
    wid = ss * NC + cc
    pltpu.sync_copy(xx_hbm, xx_v)
    pltpu.sync_copy(xy_hbm, xy_v)
    pltpu.sync_copy(xz_hbm, xz_v)
    iota = lax.iota(jnp.int32, 16)
    sidx = (sidx0, sidx1)
    didx = (didx0, didx1)
    pbuf = (pbuf0, pbuf1)
    qbuf = (qbuf0, qbuf1)
    sem_p = (sem_p0, sem_p1)
    sem_q = (sem_q0, sem_q1)
    sem_wp = (sem_wp0, sem_wp1)
    sem_wq = (sem_wq0, sem_wq1)
    niter = (NCHUNKS + NW - 1) // NW

    def drain_writes(slot):
        pltpu.make_async_copy(pbuf[slot], ps_hbm.at[pl.ds(0, CHUNK)],
                              sem_wp[slot]).wait()
        pltpu.make_async_copy(qbuf[slot], qd_hbm.at[pl.ds(0, CHUNK)],
                              sem_wq[slot]).wait()

    def fire(slot, n):
        e0 = n * CHUNK
        pltpu.sync_copy(src_hbm.at[pl.ds(e0, CHUNK)], sidx[slot])
        pltpu.sync_copy(dst_hbm.at[pl.ds(e0, CHUNK)], didx[slot])
        pltpu.async_copy(p_hbm.at[sidx[slot]], pbuf[slot], sem_p[slot])
        pltpu.async_copy(q_hbm.at[didx[slot]], qbuf[slot], sem_q[slot])

    @pl.when(wid < NCHUNKS)
    def _prologue():
        fire(0, wid)

    @pl.loop(0, (niter + 1) // 2 * 2, step=2)
    def _outer(k):
        for b in range(2):
            p, np_ = b, 1 - b
            n = wid + (k + b) * NW

            @pl.when(n < NCHUNKS)
            def _(p=p, np_=np_, n=n):
                e0 = n * CHUNK
                n_next = n + NW

                @pl.when(n_next < NCHUNKS)
                def _():
                    @pl.when(n_next >= wid + 2 * NW)
                    def _():
                        drain_writes(np_)
                    fire(np_, n_next)

                for sub in range(CHUNK // 16):
                    sv = sidx[p][pl.ds(sub * 16, 16)]
                    dv = didx[p][pl.ds(sub * 16, 16)]
                    dxv = plsc.load_gather(xx_v, [dv]) - plsc.load_gather(xx_v, [sv])
                    dyv = plsc.load_gather(xy_v, [dv]) - plsc.load_gather(xy_v, [sv])
                    dzv = plsc.load_gather(xz_v, [dv]) - plsc.load_gather(xz_v, [sv])
                    sqv = dxv * dxv + dyv * dyv + dzv * dzv
                    rows = iota + (sub * 16)
                    for comp, val in ((0, dxv), (1, dyv), (2, dzv), (3, sqv)):
                        plsc.store_scatter(
                            gbuf, [rows, jnp.full((16,), comp, jnp.int32)], val)
                pltpu.sync_copy(gbuf, geo_hbm.at[pl.ds(e0, CHUNK)])
                pltpu.make_async_copy(p_hbm.at[sidx[p]], pbuf[p], sem_p[p]).wait()
                pltpu.async_copy(pbuf[p], ps_hbm.at[pl.ds(e0, CHUNK)], sem_wp[p])
                pltpu.make_async_copy(q_hbm.at[didx[p]], qbuf[p], sem_q[p]).wait()
                pltpu.async_copy(qbuf[p], qd_hbm.at[pl.ds(e0, CHUNK)], sem_wq[p])

    drain_writes(0)
    drain_writes(1)


_CHUNKS_PER_SC = NCHUNKS // NC
_ZROWS = 16


@functools.partial(
    pl.kernel,
    out_type=jax.ShapeDtypeStruct((NC, N, H), F32),
    mesh=_MESH,
    scratch_types=[
        pltpu.VMEM_SHARED((N, H), F32),
        pltpu.VMEM((_ZROWS, H), F32),
        pltpu.VMEM((CHUNK, H), F32), pltpu.VMEM((CHUNK, H), F32),
        pltpu.VMEM((CHUNK,), jnp.int32), pltpu.VMEM((CHUNK,), jnp.int32),
        pltpu.VMEM((CHUNK,), jnp.int32), pltpu.VMEM((CHUNK,), jnp.int32),
        pltpu.SemaphoreType.DMA, pltpu.SemaphoreType.DMA,
    ],
    compiler_params=pltpu.CompilerParams(needs_layout_passes=False),
)
def _sc_scatter_m(m_hbm, src_hbm, dst_hbm, agg_out,
                  agg_sp, zbuf, mbuf0, mbuf1, sidx0, sidx1, didx0, didx1,
                  sem_m0, sem_m1):
    cc = lax.axis_index("c")
    ss = lax.axis_index("s")
    z16 = jnp.zeros((16,), F32)
    mbuf = (mbuf0, mbuf1)
    sidx = (sidx0, sidx1)
    didx = (didx0, didx1)
    sem_m = (sem_m0, sem_m1)

    @pl.loop(0, _ZROWS)
    def _zero(i):
        for j in range(H // 16):
            zbuf[i, pl.ds(j * 16, 16)] = z16

    @pl.loop(0, ROW_SPAN // _ZROWS)
    def _zfill(j):
        pltpu.sync_copy(zbuf, agg_sp.at[pl.ds(ss * ROW_STRIDE + j * _ZROWS,
                                               _ZROWS)])
    plsc.subcore_barrier()

    def fire(slot, j):
        e0 = (cc * _CHUNKS_PER_SC + j) * CHUNK
        pltpu.sync_copy(src_hbm.at[pl.ds(e0, CHUNK)], sidx[slot])
        pltpu.sync_copy(dst_hbm.at[pl.ds(e0, CHUNK)], didx[slot])
        pltpu.async_copy(m_hbm.at[pl.ds(e0, CHUNK)], mbuf[slot], sem_m[slot])

    @pl.when(ss < _CHUNKS_PER_SC)
    def _prologue():
        fire(0, ss)

    niter = (_CHUNKS_PER_SC + NS - 1) // NS

    @pl.loop(0, (niter + 1) // 2 * 2, step=2)
    def _outer(k):
        for b in range(2):
            p, np_ = b, 1 - b
            j = ss + (k + b) * NS

            @pl.when(j < _CHUNKS_PER_SC)
            def _(p=p, np_=np_, j=j):
                @pl.when(j + NS < _CHUNKS_PER_SC)
                def _():
                    fire(np_, j + NS)
                e0 = (cc * _CHUNKS_PER_SC + j) * CHUNK
                pltpu.make_async_copy(m_hbm.at[pl.ds(e0, CHUNK)],
                                      mbuf[p], sem_m[p]).wait()
                pltpu.sync_copy(mbuf[p], agg_sp.at[didx[p]], add=True)
                pltpu.sync_copy(mbuf[p], agg_sp.at[sidx[p]], add=True)

    plsc.subcore_barrier()
    r0 = ss * ROW_STRIDE
    pltpu.sync_copy(agg_sp.at[pl.ds(r0, ROW_SPAN)],
                    agg_out.at[cc, pl.ds(r0, ROW_SPAN)])


@functools.partial(
    pl.kernel,
    out_type=jax.ShapeDtypeStruct((NC, N, 16), F32),
    mesh=_MESH,
    scratch_types=[
        pltpu.VMEM_SHARED((N, 16), F32),
        pltpu.VMEM((_ZROWS, 16), F32),
        pltpu.VMEM((CHUNK, 16), F32), pltpu.VMEM((CHUNK, 16), F32),
        pltpu.VMEM((CHUNK, 16), F32), pltpu.VMEM((CHUNK, 16), F32),
        pltpu.VMEM((CHUNK,), jnp.int32), pltpu.VMEM((CHUNK,), jnp.int32),
        pltpu.VMEM((CHUNK,), jnp.int32), pltpu.VMEM((CHUNK,), jnp.int32),
        pltpu.SemaphoreType.DMA, pltpu.SemaphoreType.DMA,
        pltpu.SemaphoreType.DMA, pltpu.SemaphoreType.DMA,
    ],
    compiler_params=pltpu.CompilerParams(needs_layout_passes=False,
                                         use_tc_tiling_on_sc=False),
)
def _sc_scatter_c(cpos_hbm, cneg_hbm, src_hbm, dst_hbm, cacc_out,
                  cacc_sp, zbuf2, cbuf0, cbuf1, cbuf20, cbuf21,
                  sidx0, sidx1, didx0, didx1,
                  sem_c0, sem_c1, sem_n0, sem_n1):
    cc = lax.axis_index("c")
    ss = lax.axis_index("s")
    z16 = jnp.zeros((16,), F32)
    cbuf = (cbuf0, cbuf1)
    cbuf2 = (cbuf20, cbuf21)
    sidx = (sidx0, sidx1)
    didx = (didx0, didx1)
    sem_c = (sem_c0, sem_c1)
    sem_n = (sem_n0, sem_n1)

    @pl.loop(0, _ZROWS)
    def _zero(i):
        zbuf2[i, :] = z16

    @pl.loop(0, ROW_SPAN // _ZROWS)
    def _zfill(j):
        pltpu.sync_copy(zbuf2, cacc_sp.at[pl.ds(ss * ROW_STRIDE + j * _ZROWS,
                                                _ZROWS)])
    plsc.subcore_barrier()

    def fire(slot, j):
        e0 = (cc * _CHUNKS_PER_SC + j) * CHUNK
        pltpu.sync_copy(src_hbm.at[pl.ds(e0, CHUNK)], sidx[slot])
        pltpu.sync_copy(dst_hbm.at[pl.ds(e0, CHUNK)], didx[slot])
        pltpu.async_copy(cpos_hbm.at[pl.ds(e0, CHUNK)], cbuf[slot], sem_c[slot])
        pltpu.async_copy(cneg_hbm.at[pl.ds(e0, CHUNK)], cbuf2[slot], sem_n[slot])

    @pl.when(ss < _CHUNKS_PER_SC)
    def _prologue():
        fire(0, ss)

    niter = (_CHUNKS_PER_SC + NS - 1) // NS

    @pl.loop(0, (niter + 1) // 2 * 2, step=2)
    def _outer(k):
        for b in range(2):
            p, np_ = b, 1 - b
            j = ss + (k + b) * NS

            @pl.when(j < _CHUNKS_PER_SC)
            def _(p=p, np_=np_, j=j):
                @pl.when(j + NS < _CHUNKS_PER_SC)
                def _():
                    fire(np_, j + NS)
                e0 = (cc * _CHUNKS_PER_SC + j) * CHUNK
                pltpu.make_async_copy(cpos_hbm.at[pl.ds(e0, CHUNK)],
                                      cbuf[p], sem_c[p]).wait()
                pltpu.sync_copy(cbuf[p], cacc_sp.at[didx[p]], add=True)
                pltpu.make_async_copy(cneg_hbm.at[pl.ds(e0, CHUNK)],
                                      cbuf2[p], sem_n[p]).wait()
                pltpu.sync_copy(cbuf2[p], cacc_sp.at[sidx[p]], add=True)

    plsc.subcore_barrier()
    r0 = ss * ROW_STRIDE
    pltpu.sync_copy(cacc_sp.at[pl.ds(r0, ROW_SPAN)],
                    cacc_out.at[cc, pl.ds(r0, ROW_SPAN)])


_BN = 1000
_BE = 2000


def kernel(h, x, bond_indices, W1, b1, W2, b2, Wn1, bn1, Wn2, bn2,
           Wc1, bc1, Wc2):
    h2 = h[0]
    x2 = x[0]
    src = bond_indices[:, 0].astype(jnp.int32)
    dst = bond_indices[:, 1].astype(jnp.int32)
    xx, xy, xz = x2[:, 0], x2[:, 1], x2[:, 2]
    w1a, w1b, w1c = W1[:H], W1[H:2 * H], W1[2 * H:2 * H + 1]
    b1r, b2r, bc1r = b1[None], b2[None], bc1[None]
    wn1a, wn1b = Wn1[:H], Wn1[H:2 * H]
    bn1r, bn2r = bn1[None], bn2[None]

    p_arr, q_arr = pl.pallas_call(
        _b0_body,
        grid=(N // _BN,),
        in_specs=[pl.BlockSpec((_BN, H), lambda i: (i, 0)),
                  pl.BlockSpec((H, H), lambda i: (0, 0)),
                  pl.BlockSpec((H, H), lambda i: (0, 0))],
        out_specs=[pl.BlockSpec((_BN, H), lambda i: (i, 0)),
                   pl.BlockSpec((_BN, H), lambda i: (i, 0))],
        out_shape=[jax.ShapeDtypeStruct((N, H), F32),
                   jax.ShapeDtypeStruct((N, H), F32)],
    )(h2, w1a, w1b)

    ps, qd, geo = _sc_gather(p_arr, q_arr, src, dst, xx, xy, xz)

    m_arr, cpos, cneg = pl.pallas_call(
        _edge_body,
        grid=(E // _BE,),
        in_specs=[pl.BlockSpec((_BE, H), lambda i: (i, 0)),
                  pl.BlockSpec((_BE, H), lambda i: (i, 0)),
                  pl.BlockSpec((_BE, 4), lambda i: (i, 0)),
                  pl.BlockSpec((1, H), lambda i: (0, 0)),
                  pl.BlockSpec((1, H), lambda i: (0, 0)),
                  pl.BlockSpec((H, H), lambda i: (0, 0)),
                  pl.BlockSpec((1, H), lambda i: (0, 0)),
                  pl.BlockSpec((H, H), lambda i: (0, 0)),
                  pl.BlockSpec((1, H), lambda i: (0, 0)),
                  pl.BlockSpec((H, 1), lambda i: (0, 0))],
        out_specs=[pl.BlockSpec((_BE, H), lambda i: (i, 0)),
                   pl.BlockSpec((_BE, 16), lambda i: (i, 0)),
                   pl.BlockSpec((_BE, 16), lambda i: (i, 0))],
        out_shape=[jax.ShapeDtypeStruct((E, H), F32),
                   jax.ShapeDtypeStruct((E, 16), F32),
                   jax.ShapeDtypeStruct((E, 16), F32)],
    )(ps, qd, geo, w1c, b1r, W2, b2r, Wc1, bc1r, Wc2)

    aggp = _sc_scatter_m(m_arr, src, dst)
    caccp = _sc_scatter_c(cpos, cneg, src, dst)

    h_out, x_out = pl.pallas_call(
        _node_body,
        grid=(N // _BN,),
        in_specs=[pl.BlockSpec((_BN, H), lambda i: (i, 0)),
                  pl.BlockSpec((_BN, 3), lambda i: (i, 0)),
                  pl.BlockSpec((NC, _BN, H), lambda i: (0, i, 0)),
                  pl.BlockSpec((NC, _BN, 16), lambda i: (0, i, 0)),
                  pl.BlockSpec((H, H), lambda i: (0, 0)),
                  pl.BlockSpec((H, H), lambda i: (0, 0)),
                  pl.BlockSpec((1, H), lambda i: (0, 0)),
                  pl.BlockSpec((H, H), lambda i: (0, 0)),
                  pl.BlockSpec((1, H), lambda i: (0, 0))],
        out_specs=[pl.BlockSpec((_BN, H), lambda i: (i, 0)),
                   pl.BlockSpec((_BN, 3), lambda i: (i, 0))],
        out_shape=[jax.ShapeDtypeStruct((N, H), F32),
                   jax.ShapeDtypeStruct((N, 3), F32)],
    )(h2, x2, aggp, caccp, wn1a, wn1b, bn1r, Wn2, bn2r)

    return h_out[None], x_out[None]


# speedup vs baseline: 2.3489x; 1.0346x over previous
import functools

import jax
import jax.numpy as jnp
from jax import lax
from jax.experimental import pallas as pl
from jax.experimental.pallas import tpu as pltpu
from jax.experimental.pallas import tpu_sc as plsc

N = 10000
H = 128
E = 320000
CHUNK = 128
NCHUNKS = E // CHUNK
NC, NS, NW = 2, 16, 32
ROW_STRIDE = 624
ROW_SPAN = 640
F32 = jnp.float32


def _b0_body(h_ref, wa_ref, wb_ref, p_ref, q_ref):
    hb = h_ref[...]
    p_ref[...] = jnp.dot(hb, wa_ref[...], preferred_element_type=F32)
    q_ref[...] = jnp.dot(hb, wb_ref[...], preferred_element_type=F32)


def _edge_body(ps_ref, qd_ref, geo_ref, w1c_ref, b1_ref, w2_ref, b2_ref,
               wc1_ref, bc1_ref, wc2_ref, m_ref, cpos_ref, cneg_ref):
    g = geo_ref[...]
    dist = jnp.sqrt(g[:, 3:4])
    pre = ps_ref[...] + qd_ref[...] + dist * w1c_ref[...] + b1_ref[...]
    t1 = jax.nn.silu(pre)
    mm = jax.nn.silu(jnp.dot(t1, w2_ref[...], preferred_element_type=F32)
                     + b2_ref[...])
    m_ref[...] = mm
    t2 = jax.nn.silu(jnp.dot(mm, wc1_ref[...], preferred_element_type=F32)
                     + bc1_ref[...])
    cw = jnp.dot(t2, wc2_ref[...], preferred_element_type=F32)
    s = cw / (dist + 1e-8)
    cvec = g[:, 0:3] * s
    ones = jnp.ones_like(cw)
    zeros = jnp.zeros(cvec.shape[:1] + (12,), F32)
    cpos_ref[...] = jnp.concatenate([cvec, ones, zeros], axis=1)
    cneg_ref[...] = jnp.concatenate([-cvec, ones, zeros], axis=1)


def _node_body(h_ref, x_ref, agg_ref, cacc_ref, wn1a_ref, wn1b_ref, bn1_ref,
               wn2_ref, bn2_ref, ho_ref, xo_ref):
    a3 = agg_ref[...]
    a = a3[0] + a3[1]
    hb = h_ref[...]
    t = jax.nn.silu(jnp.dot(hb, wn1a_ref[...], preferred_element_type=F32)
                    + jnp.dot(a, wn1b_ref[...], preferred_element_type=F32)
                    + bn1_ref[...])
    ho_ref[...] = hb + jnp.dot(t, wn2_ref[...], preferred_element_type=F32) \
        + bn2_ref[...]
    c3 = cacc_ref[...]
    c = c3[0] + c3[1]
    cnt = jnp.maximum(c[:, 3:4], 1.0)
    xo_ref[...] = x_ref[...] + c[:, 0:3] / cnt


_MESH = plsc.VectorSubcoreMesh(core_axis_name="c", subcore_axis_name="s")


@functools.partial(
    pl.kernel,
    out_type=(jax.ShapeDtypeStruct((E, H), F32),
              jax.ShapeDtypeStruct((E, H), F32),
              jax.ShapeDtypeStruct((E, 4), F32)),
    mesh=_MESH,
    scratch_types=[
        pltpu.VMEM((N,), F32), pltpu.VMEM((N,), F32), pltpu.VMEM((N,), F32),
        pltpu.VMEM((CHUNK,), jnp.int32), pltpu.VMEM((CHUNK,), jnp.int32),
        pltpu.VMEM((CHUNK,), jnp.int32), pltpu.VMEM((CHUNK,), jnp.int32),
        pltpu.VMEM((CHUNK, H), F32), pltpu.VMEM((CHUNK, H), F32),
        pltpu.VMEM((CHUNK, H), F32), pltpu.VMEM((CHUNK, H), F32),
        pltpu.VMEM((CHUNK, 4), F32),
        pltpu.SemaphoreType.DMA, pltpu.SemaphoreType.DMA,
        pltpu.SemaphoreType.DMA, pltpu.SemaphoreType.DMA,
        pltpu.SemaphoreType.DMA, pltpu.SemaphoreType.DMA,
        pltpu.SemaphoreType.DMA, pltpu.SemaphoreType.DMA,
    ],
    compiler_params=pltpu.CompilerParams(needs_layout_passes=False),
)
def _sc_gather(p_hbm, q_hbm, src_hbm, dst_hbm, xx_hbm, xy_hbm, xz_hbm,
               ps_hbm, qd_hbm, geo_hbm,
               xx_v, xy_v, xz_v, sidx0, sidx1, didx0, didx1,
               pbuf0, pbuf1, qbuf0, qbuf1, gbuf,
               sem_p0, sem_p1, sem_q0, sem_q1,
               sem_wp0, sem_wp1, sem_wq0, sem_wq1):
    cc = lax.axis_index("c")
    ss = lax.axis_index("s")
    wid = ss * NC + cc
    pltpu.sync_copy(xx_hbm, xx_v)
    pltpu.sync_copy(xy_hbm, xy_v)
    pltpu.sync_copy(xz_hbm, xz_v)
    iota = lax.iota(jnp.int32, 16)
    sidx = (sidx0, sidx1)
    didx = (didx0, didx1)
    pbuf = (pbuf0, pbuf1)
    qbuf = (qbuf0, qbuf1)
    sem_p = (sem_p0, sem_p1)
    sem_q = (sem_q0, sem_q1)
    sem_wp = (sem_wp0, sem_wp1)
    sem_wq = (sem_wq0, sem_wq1)
    niter = (NCHUNKS + NW - 1) // NW

    def drain_writes(slot):
        pltpu.make_async_copy(pbuf[slot], ps_hbm.at[pl.ds(0, CHUNK)],
                              sem_wp[slot]).wait()
        pltpu.make_async_copy(qbuf[slot], qd_hbm.at[pl.ds(0, CHUNK)],
                              sem_wq[slot]).wait()

    def fire(slot, n):
        e0 = n * CHUNK
        pltpu.sync_copy(src_hbm.at[pl.ds(e0, CHUNK)], sidx[slot])
        pltpu.sync_copy(dst_hbm.at[pl.ds(e0, CHUNK)], didx[slot])
        pltpu.async_copy(p_hbm.at[sidx[slot]], pbuf[slot], sem_p[slot])
        pltpu.async_copy(q_hbm.at[didx[slot]], qbuf[slot], sem_q[slot])

    @pl.when(wid < NCHUNKS)
    def _prologue():
        fire(0, wid)

    @pl.loop(0, (niter + 1) // 2 * 2, step=2)
    def _outer(k):
        for b in range(2):
            p, np_ = b, 1 - b
            n = wid + (k + b) * NW

            @pl.when(n < NCHUNKS)
            def _(p=p, np_=np_, n=n):
                e0 = n * CHUNK
                n_next = n + NW

                @pl.when(n_next < NCHUNKS)
                def _():
                    @pl.when(n_next >= wid + 2 * NW)
                    def _():
                        drain_writes(np_)
                    fire(np_, n_next)

                for sub in range(CHUNK // 16):
                    sv = sidx[p][pl.ds(sub * 16, 16)]
                    dv = didx[p][pl.ds(sub * 16, 16)]
                    dxv = plsc.load_gather(xx_v, [dv]) - plsc.load_gather(xx_v, [sv])
                    dyv = plsc.load_gather(xy_v, [dv]) - plsc.load_gather(xy_v, [sv])
                    dzv = plsc.load_gather(xz_v, [dv]) - plsc.load_gather(xz_v, [sv])
                    sqv = dxv * dxv + dyv * dyv + dzv * dzv
                    rows = iota + (sub * 16)
                    for comp, val in ((0, dxv), (1, dyv), (2, dzv), (3, sqv)):
                        plsc.store_scatter(
                            gbuf, [rows, jnp.full((16,), comp, jnp.int32)], val)
                pltpu.sync_copy(gbuf, geo_hbm.at[pl.ds(e0, CHUNK)])
                pltpu.make_async_copy(p_hbm.at[sidx[p]], pbuf[p], sem_p[p]).wait()
                pltpu.async_copy(pbuf[p], ps_hbm.at[pl.ds(e0, CHUNK)], sem_wp[p])
                pltpu.make_async_copy(q_hbm.at[didx[p]], qbuf[p], sem_q[p]).wait()
                pltpu.async_copy(qbuf[p], qd_hbm.at[pl.ds(e0, CHUNK)], sem_wq[p])

    drain_writes(0)
    drain_writes(1)


_CHUNKS_PER_SC = NCHUNKS // NC
_ZROWS = 16


@functools.partial(
    pl.kernel,
    out_type=jax.ShapeDtypeStruct((NC, N, H), F32),
    mesh=_MESH,
    scratch_types=[
        pltpu.VMEM_SHARED((N, H), F32),
        pltpu.VMEM((_ZROWS, H), F32),
        pltpu.VMEM((CHUNK, H), F32), pltpu.VMEM((CHUNK, H), F32),
        pltpu.VMEM((CHUNK,), jnp.int32), pltpu.VMEM((CHUNK,), jnp.int32),
        pltpu.VMEM((CHUNK,), jnp.int32), pltpu.VMEM((CHUNK,), jnp.int32),
        pltpu.SemaphoreType.DMA, pltpu.SemaphoreType.DMA,
    ],
    compiler_params=pltpu.CompilerParams(needs_layout_passes=False),
)
def _sc_scatter_m(m_hbm, src_hbm, dst_hbm, agg_out,
                  agg_sp, zbuf, mbuf0, mbuf1, sidx0, sidx1, didx0, didx1,
                  sem_m0, sem_m1):
    cc = lax.axis_index("c")
    ss = lax.axis_index("s")
    z16 = jnp.zeros((16,), F32)
    mbuf = (mbuf0, mbuf1)
    sidx = (sidx0, sidx1)
    didx = (didx0, didx1)
    sem_m = (sem_m0, sem_m1)

    @pl.loop(0, _ZROWS)
    def _zero(i):
        for j in range(H // 16):
            zbuf[i, pl.ds(j * 16, 16)] = z16

    @pl.loop(0, ROW_SPAN // _ZROWS)
    def _zfill(j):
        pltpu.sync_copy(zbuf, agg_sp.at[pl.ds(ss * ROW_STRIDE + j * _ZROWS,
                                               _ZROWS)])
    plsc.subcore_barrier()

    def fire(slot, j):
        e0 = (cc * _CHUNKS_PER_SC + j) * CHUNK
        pltpu.sync_copy(src_hbm.at[pl.ds(e0, CHUNK)], sidx[slot])
        pltpu.sync_copy(dst_hbm.at[pl.ds(e0, CHUNK)], didx[slot])
        pltpu.async_copy(m_hbm.at[pl.ds(e0, CHUNK)], mbuf[slot], sem_m[slot])

    @pl.when(ss < _CHUNKS_PER_SC)
    def _prologue():
        fire(0, ss)

    niter = (_CHUNKS_PER_SC + NS - 1) // NS

    @pl.loop(0, (niter + 1) // 2 * 2, step=2)
    def _outer(k):
        for b in range(2):
            p, np_ = b, 1 - b
            j = ss + (k + b) * NS

            @pl.when(j < _CHUNKS_PER_SC)
            def _(p=p, np_=np_, j=j):
                @pl.when(j + NS < _CHUNKS_PER_SC)
                def _():
                    fire(np_, j + NS)
                e0 = (cc * _CHUNKS_PER_SC + j) * CHUNK
                pltpu.make_async_copy(m_hbm.at[pl.ds(e0, CHUNK)],
                                      mbuf[p], sem_m[p]).wait()
                pltpu.sync_copy(mbuf[p], agg_sp.at[didx[p]], add=True)
                pltpu.sync_copy(mbuf[p], agg_sp.at[sidx[p]], add=True)

    plsc.subcore_barrier()
    r0 = ss * ROW_STRIDE
    pltpu.sync_copy(agg_sp.at[pl.ds(r0, ROW_SPAN)],
                    agg_out.at[cc, pl.ds(r0, ROW_SPAN)])


@functools.partial(
    pl.kernel,
    out_type=jax.ShapeDtypeStruct((NC, N, 16), F32),
    mesh=_MESH,
    scratch_types=[
        pltpu.VMEM_SHARED((N, 16), F32),
        pltpu.VMEM((_ZROWS, 16), F32),
        pltpu.VMEM((CHUNK, 16), F32), pltpu.VMEM((CHUNK, 16), F32),
        pltpu.VMEM((CHUNK, 16), F32), pltpu.VMEM((CHUNK, 16), F32),
        pltpu.VMEM((CHUNK,), jnp.int32), pltpu.VMEM((CHUNK,), jnp.int32),
        pltpu.VMEM((CHUNK,), jnp.int32), pltpu.VMEM((CHUNK,), jnp.int32),
        pltpu.SemaphoreType.DMA, pltpu.SemaphoreType.DMA,
        pltpu.SemaphoreType.DMA, pltpu.SemaphoreType.DMA,
    ],
    compiler_params=pltpu.CompilerParams(needs_layout_passes=False,
                                         use_tc_tiling_on_sc=False),
)
def _sc_scatter_c(cpos_hbm, cneg_hbm, src_hbm, dst_hbm, cacc_out,
                  cacc_sp, zbuf2, cbuf0, cbuf1, cbuf20, cbuf21,
                  sidx0, sidx1, didx0, didx1,
                  sem_c0, sem_c1, sem_n0, sem_n1):
    cc = lax.axis_index("c")
    ss = lax.axis_index("s")
    z16 = jnp.zeros((16,), F32)
    cbuf = (cbuf0, cbuf1)
    cbuf2 = (cbuf20, cbuf21)
    sidx = (sidx0, sidx1)
    didx = (didx0, didx1)
    sem_c = (sem_c0, sem_c1)
    sem_n = (sem_n0, sem_n1)

    @pl.loop(0, _ZROWS)
    def _zero(i):
        zbuf2[i, :] = z16

    @pl.loop(0, ROW_SPAN // _ZROWS)
    def _zfill(j):
        pltpu.sync_copy(zbuf2, cacc_sp.at[pl.ds(ss * ROW_STRIDE + j * _ZROWS,
                                                _ZROWS)])
    plsc.subcore_barrier()

    def fire(slot, j):
        e0 = (cc * _CHUNKS_PER_SC + j) * CHUNK
        pltpu.sync_copy(src_hbm.at[pl.ds(e0, CHUNK)], sidx[slot])
        pltpu.sync_copy(dst_hbm.at[pl.ds(e0, CHUNK)], didx[slot])
        pltpu.async_copy(cpos_hbm.at[pl.ds(e0, CHUNK)], cbuf[slot], sem_c[slot])
        pltpu.async_copy(cneg_hbm.at[pl.ds(e0, CHUNK)], cbuf2[slot], sem_n[slot])

    @pl.when(ss < _CHUNKS_PER_SC)
    def _prologue():
        fire(0, ss)

    niter = (_CHUNKS_PER_SC + NS - 1) // NS

    @pl.loop(0, (niter + 1) // 2 * 2, step=2)
    def _outer(k):
        for b in range(2):
            p, np_ = b, 1 - b
            j = ss + (k + b) * NS

            @pl.when(j < _CHUNKS_PER_SC)
            def _(p=p, np_=np_, j=j):
                @pl.when(j + NS < _CHUNKS_PER_SC)
                def _():
                    fire(np_, j + NS)
                e0 = (cc * _CHUNKS_PER_SC + j) * CHUNK
                pltpu.make_async_copy(cpos_hbm.at[pl.ds(e0, CHUNK)],
                                      cbuf[p], sem_c[p]).wait()
                pltpu.sync_copy(cbuf[p], cacc_sp.at[didx[p]], add=True)
                pltpu.make_async_copy(cneg_hbm.at[pl.ds(e0, CHUNK)],
                                      cbuf2[p], sem_n[p]).wait()
                pltpu.sync_copy(cbuf2[p], cacc_sp.at[sidx[p]], add=True)

    plsc.subcore_barrier()
    r0 = ss * ROW_STRIDE
    pltpu.sync_copy(cacc_sp.at[pl.ds(r0, ROW_SPAN)],
                    cacc_out.at[cc, pl.ds(r0, ROW_SPAN)])


_BN = 1000
_BE = 4000


def kernel(h, x, bond_indices, W1, b1, W2, b2, Wn1, bn1, Wn2, bn2,
           Wc1, bc1, Wc2):
    h2 = h[0]
    x2 = x[0]
    src = bond_indices[:, 0].astype(jnp.int32)
    dst = bond_indices[:, 1].astype(jnp.int32)
    xx, xy, xz = x2[:, 0], x2[:, 1], x2[:, 2]
    w1a, w1b, w1c = W1[:H], W1[H:2 * H], W1[2 * H:2 * H + 1]
    b1r, b2r, bc1r = b1[None], b2[None], bc1[None]
    wn1a, wn1b = Wn1[:H], Wn1[H:2 * H]
    bn1r, bn2r = bn1[None], bn2[None]

    p_arr, q_arr = pl.pallas_call(
        _b0_body,
        grid=(N // _BN,),
        in_specs=[pl.BlockSpec((_BN, H), lambda i: (i, 0)),
                  pl.BlockSpec((H, H), lambda i: (0, 0)),
                  pl.BlockSpec((H, H), lambda i: (0, 0))],
        out_specs=[pl.BlockSpec((_BN, H), lambda i: (i, 0)),
                   pl.BlockSpec((_BN, H), lambda i: (i, 0))],
        out_shape=[jax.ShapeDtypeStruct((N, H), F32),
                   jax.ShapeDtypeStruct((N, H), F32)],
    )(h2, w1a, w1b)

    ps, qd, geo = _sc_gather(p_arr, q_arr, src, dst, xx, xy, xz)

    m_arr, cpos, cneg = pl.pallas_call(
        _edge_body,
        grid=(E // _BE,),
        in_specs=[pl.BlockSpec((_BE, H), lambda i: (i, 0)),
                  pl.BlockSpec((_BE, H), lambda i: (i, 0)),
                  pl.BlockSpec((_BE, 4), lambda i: (i, 0)),
                  pl.BlockSpec((1, H), lambda i: (0, 0)),
                  pl.BlockSpec((1, H), lambda i: (0, 0)),
                  pl.BlockSpec((H, H), lambda i: (0, 0)),
                  pl.BlockSpec((1, H), lambda i: (0, 0)),
                  pl.BlockSpec((H, H), lambda i: (0, 0)),
                  pl.BlockSpec((1, H), lambda i: (0, 0)),
                  pl.BlockSpec((H, 1), lambda i: (0, 0))],
        out_specs=[pl.BlockSpec((_BE, H), lambda i: (i, 0)),
                   pl.BlockSpec((_BE, 16), lambda i: (i, 0)),
                   pl.BlockSpec((_BE, 16), lambda i: (i, 0))],
        out_shape=[jax.ShapeDtypeStruct((E, H), F32),
                   jax.ShapeDtypeStruct((E, 16), F32),
                   jax.ShapeDtypeStruct((E, 16), F32)],
    )(ps, qd, geo, w1c, b1r, W2, b2r, Wc1, bc1r, Wc2)

    aggp = _sc_scatter_m(m_arr, src, dst)
    caccp = _sc_scatter_c(cpos, cneg, src, dst)

    h_out, x_out = pl.pallas_call(
        _node_body,
        grid=(N // _BN,),
        in_specs=[pl.BlockSpec((_BN, H), lambda i: (i, 0)),
                  pl.BlockSpec((_BN, 3), lambda i: (i, 0)),
                  pl.BlockSpec((NC, _BN, H), lambda i: (0, i, 0)),
                  pl.BlockSpec((NC, _BN, 16), lambda i: (0, i, 0)),
                  pl.BlockSpec((H, H), lambda i: (0, 0)),
                  pl.BlockSpec((H, H), lambda i: (0, 0)),
                  pl.BlockSpec((1, H), lambda i: (0, 0)),
                  pl.BlockSpec((H, H), lambda i: (0, 0)),
                  pl.BlockSpec((1, H), lambda i: (0, 0))],
        out_specs=[pl.BlockSpec((_BN, H), lambda i: (i, 0)),
                   pl.BlockSpec((_BN, 3), lambda i: (i, 0))],
        out_shape=[jax.ShapeDtypeStruct((N, H), F32),
                   jax.ShapeDtypeStruct((N, 3), F32)],
    )(h2, x2, aggp, caccp, wn1a, wn1b, bn1r, Wn2, bn2r)

    return h_out[None], x_out[None]


# speedup vs baseline: 2.3720x; 1.0098x over previous
import functools

import jax
import jax.numpy as jnp
from jax import lax
from jax.experimental import pallas as pl
from jax.experimental.pallas import tpu as pltpu
from jax.experimental.pallas import tpu_sc as plsc

N = 10000
H = 128
E = 320000
CHUNK = 128
NCHUNKS = E // CHUNK
NC, NS, NW = 2, 16, 32
ROW_STRIDE = 624
ROW_SPAN = 640
F32 = jnp.float32


def _b0_body(h_ref, wa_ref, wb_ref, p_ref, q_ref):
    hb = h_ref[...]
    p_ref[...] = jnp.dot(hb, wa_ref[...], preferred_element_type=F32)
    q_ref[...] = jnp.dot(hb, wb_ref[...], preferred_element_type=F32)


def _edge_body(ps_ref, qd_ref, geo_ref, w1c_ref, b1_ref, w2_ref, b2_ref,
               wc1_ref, bc1_ref, wc2_ref, m_ref, cpos_ref, cneg_ref):
    g = geo_ref[...]
    dist = jnp.sqrt(g[:, 3:4])
    pre = ps_ref[...] + qd_ref[...] + dist * w1c_ref[...] + b1_ref[...]
    t1 = jax.nn.silu(pre)
    mm = jax.nn.silu(jnp.dot(t1, w2_ref[...], preferred_element_type=F32)
                     + b2_ref[...])
    m_ref[...] = mm
    t2 = jax.nn.silu(jnp.dot(mm, wc1_ref[...], preferred_element_type=F32)
                     + bc1_ref[...])
    cw = jnp.dot(t2, wc2_ref[...], preferred_element_type=F32)
    s = cw / (dist + 1e-8)
    cvec = g[:, 0:3] * s
    ones = jnp.ones_like(cw)
    zeros = jnp.zeros(cvec.shape[:1] + (12,), F32)
    cpos_ref[...] = jnp.concatenate([cvec, ones, zeros], axis=1)
    cneg_ref[...] = jnp.concatenate([-cvec, ones, zeros], axis=1)


def _node_body(h_ref, x_ref, agg_ref, cacc_ref, wn1a_ref, wn1b_ref, bn1_ref,
               wn2_ref, bn2_ref, ho_ref, xo_ref):
    a3 = agg_ref[...]
    a = a3[0] + a3[1]
    hb = h_ref[...]
    t = jax.nn.silu(jnp.dot(hb, wn1a_ref[...], preferred_element_type=F32)
                    + jnp.dot(a, wn1b_ref[...], preferred_element_type=F32)
                    + bn1_ref[...])
    ho_ref[...] = hb + jnp.dot(t, wn2_ref[...], preferred_element_type=F32) \
        + bn2_ref[...]
    c3 = cacc_ref[...]
    c = c3[0] + c3[1]
    cnt = jnp.maximum(c[:, 3:4], 1.0)
    xo_ref[...] = x_ref[...] + c[:, 0:3] / cnt


_MESH = plsc.VectorSubcoreMesh(core_axis_name="c", subcore_axis_name="s")


@functools.partial(
    pl.kernel,
    out_type=(jax.ShapeDtypeStruct((E, H), F32),
              jax.ShapeDtypeStruct((E, H), F32),
              jax.ShapeDtypeStruct((E, 4), F32)),
    mesh=_MESH,
    scratch_types=[
        pltpu.VMEM((N,), F32), pltpu.VMEM((N,), F32), pltpu.VMEM((N,), F32),
        pltpu.VMEM((CHUNK,), jnp.int32), pltpu.VMEM((CHUNK,), jnp.int32),
        pltpu.VMEM((CHUNK,), jnp.int32), pltpu.VMEM((CHUNK,), jnp.int32),
        pltpu.VMEM((CHUNK, H), F32), pltpu.VMEM((CHUNK, H), F32),
        pltpu.VMEM((CHUNK, H), F32), pltpu.VMEM((CHUNK, H), F32),
        pltpu.VMEM((CHUNK, 4), F32),
        pltpu.SemaphoreType.DMA, pltpu.SemaphoreType.DMA,
        pltpu.SemaphoreType.DMA, pltpu.SemaphoreType.DMA,
        pltpu.SemaphoreType.DMA, pltpu.SemaphoreType.DMA,
        pltpu.SemaphoreType.DMA, pltpu.SemaphoreType.DMA,
    ],
    compiler_params=pltpu.CompilerParams(needs_layout_passes=False),
)
def _sc_gather(p_hbm, q_hbm, src_hbm, dst_hbm, xx_hbm, xy_hbm, xz_hbm,
               ps_hbm, qd_hbm, geo_hbm,
               xx_v, xy_v, xz_v, sidx0, sidx1, didx0, didx1,
               pbuf0, pbuf1, qbuf0, qbuf1, gbuf,
               sem_p0, sem_p1, sem_q0, sem_q1,
               sem_wp0, sem_wp1, sem_wq0, sem_wq1):
    cc = lax.axis_index("c")
    ss = lax.axis_index("s")
    wid = ss * NC + cc
    pltpu.sync_copy(xx_hbm, xx_v)
    pltpu.sync_copy(xy_hbm, xy_v)
    pltpu.sync_copy(xz_hbm, xz_v)
    iota = lax.iota(jnp.int32, 16)
    sidx = (sidx0, sidx1)
    didx = (didx0, didx1)
    pbuf = (pbuf0, pbuf1)
    qbuf = (qbuf0, qbuf1)
    sem_p = (sem_p0, sem_p1)
    sem_q = (sem_q0, sem_q1)
    sem_wp = (sem_wp0, sem_wp1)
    sem_wq = (sem_wq0, sem_wq1)
    niter = (NCHUNKS + NW - 1) // NW

    def drain_writes(slot):
        pltpu.make_async_copy(pbuf[slot], ps_hbm.at[pl.ds(0, CHUNK)],
                              sem_wp[slot]).wait()
        pltpu.make_async_copy(qbuf[slot], qd_hbm.at[pl.ds(0, CHUNK)],
                              sem_wq[slot]).wait()

    def fire(slot, n):
        e0 = n * CHUNK
        pltpu.sync_copy(src_hbm.at[pl.ds(e0, CHUNK)], sidx[slot])
        pltpu.sync_copy(dst_hbm.at[pl.ds(e0, CHUNK)], didx[slot])
        pltpu.async_copy(p_hbm.at[sidx[slot]], pbuf[slot], sem_p[slot])
        pltpu.async_copy(q_hbm.at[didx[slot]], qbuf[slot], sem_q[slot])

    @pl.when(wid < NCHUNKS)
    def _prologue():
        fire(0, wid)

    @pl.loop(0, (niter + 1) // 2 * 2, step=2)
    def _outer(k):
        for b in range(2):
            p, np_ = b, 1 - b
            n = wid + (k + b) * NW

            @pl.when(n < NCHUNKS)
            def _(p=p, np_=np_, n=n):
                e0 = n * CHUNK
                n_next = n + NW

                @pl.when(n_next < NCHUNKS)
                def _():
                    @pl.when(n_next >= wid + 2 * NW)
                    def _():
                        drain_writes(np_)
                    fire(np_, n_next)

                for sub in range(CHUNK // 16):
                    sv = sidx[p][pl.ds(sub * 16, 16)]
                    dv = didx[p][pl.ds(sub * 16, 16)]
                    dxv = plsc.load_gather(xx_v, [dv]) - plsc.load_gather(xx_v, [sv])
                    dyv = plsc.load_gather(xy_v, [dv]) - plsc.load_gather(xy_v, [sv])
                    dzv = plsc.load_gather(xz_v, [dv]) - plsc.load_gather(xz_v, [sv])
                    sqv = dxv * dxv + dyv * dyv + dzv * dzv
                    rows = iota + (sub * 16)
                    for comp, val in ((0, dxv), (1, dyv), (2, dzv), (3, sqv)):
                        plsc.store_scatter(
                            gbuf, [rows, jnp.full((16,), comp, jnp.int32)], val)
                pltpu.sync_copy(gbuf, geo_hbm.at[pl.ds(e0, CHUNK)])
                pltpu.make_async_copy(p_hbm.at[sidx[p]], pbuf[p], sem_p[p]).wait()
                pltpu.async_copy(pbuf[p], ps_hbm.at[pl.ds(e0, CHUNK)], sem_wp[p])
                pltpu.make_async_copy(q_hbm.at[didx[p]], qbuf[p], sem_q[p]).wait()
                pltpu.async_copy(qbuf[p], qd_hbm.at[pl.ds(e0, CHUNK)], sem_wq[p])

    drain_writes(0)
    drain_writes(1)


_CHUNKS_PER_SC = NCHUNKS // NC
_ZROWS = 16


@functools.partial(
    pl.kernel,
    out_type=jax.ShapeDtypeStruct((NC, N, H), F32),
    mesh=_MESH,
    scratch_types=[
        pltpu.VMEM_SHARED((N, H), F32),
        pltpu.VMEM((_ZROWS, H), F32),
        pltpu.VMEM((CHUNK, H), F32), pltpu.VMEM((CHUNK, H), F32),
        pltpu.VMEM((CHUNK,), jnp.int32), pltpu.VMEM((CHUNK,), jnp.int32),
        pltpu.VMEM((CHUNK,), jnp.int32), pltpu.VMEM((CHUNK,), jnp.int32),
        pltpu.SemaphoreType.DMA, pltpu.SemaphoreType.DMA,
    ],
    compiler_params=pltpu.CompilerParams(needs_layout_passes=False),
)
def _sc_scatter_m(m_hbm, src_hbm, dst_hbm, agg_out,
                  agg_sp, zbuf, mbuf0, mbuf1, sidx0, sidx1, didx0, didx1,
                  sem_m0, sem_m1):
    cc = lax.axis_index("c")
    ss = lax.axis_index("s")
    z16 = jnp.zeros((16,), F32)
    mbuf = (mbuf0, mbuf1)
    sidx = (sidx0, sidx1)
    didx = (didx0, didx1)
    sem_m = (sem_m0, sem_m1)

    @pl.loop(0, _ZROWS)
    def _zero(i):
        for j in range(H // 16):
            zbuf[i, pl.ds(j * 16, 16)] = z16

    @pl.loop(0, ROW_SPAN // _ZROWS)
    def _zfill(j):
        pltpu.sync_copy(zbuf, agg_sp.at[pl.ds(ss * ROW_STRIDE + j * _ZROWS,
                                               _ZROWS)])
    plsc.subcore_barrier()

    def fire(slot, j):
        e0 = (cc * _CHUNKS_PER_SC + j) * CHUNK
        pltpu.sync_copy(src_hbm.at[pl.ds(e0, CHUNK)], sidx[slot])
        pltpu.sync_copy(dst_hbm.at[pl.ds(e0, CHUNK)], didx[slot])
        pltpu.async_copy(m_hbm.at[pl.ds(e0, CHUNK)], mbuf[slot], sem_m[slot])

    @pl.when(ss < _CHUNKS_PER_SC)
    def _prologue():
        fire(0, ss)

    niter = (_CHUNKS_PER_SC + NS - 1) // NS

    @pl.loop(0, (niter + 1) // 2 * 2, step=2)
    def _outer(k):
        for b in range(2):
            p, np_ = b, 1 - b
            j = ss + (k + b) * NS

            @pl.when(j < _CHUNKS_PER_SC)
            def _(p=p, np_=np_, j=j):
                @pl.when(j + NS < _CHUNKS_PER_SC)
                def _():
                    fire(np_, j + NS)
                e0 = (cc * _CHUNKS_PER_SC + j) * CHUNK
                pltpu.make_async_copy(m_hbm.at[pl.ds(e0, CHUNK)],
                                      mbuf[p], sem_m[p]).wait()
                pltpu.sync_copy(mbuf[p], agg_sp.at[didx[p]], add=True)
                pltpu.sync_copy(mbuf[p], agg_sp.at[sidx[p]], add=True)

    plsc.subcore_barrier()
    r0 = ss * ROW_STRIDE
    pltpu.sync_copy(agg_sp.at[pl.ds(r0, ROW_SPAN)],
                    agg_out.at[cc, pl.ds(r0, ROW_SPAN)])


@functools.partial(
    pl.kernel,
    out_type=jax.ShapeDtypeStruct((NC, N, 16), F32),
    mesh=_MESH,
    scratch_types=[
        pltpu.VMEM_SHARED((N, 16), F32),
        pltpu.VMEM((_ZROWS, 16), F32),
        pltpu.VMEM((CHUNK, 16), F32), pltpu.VMEM((CHUNK, 16), F32),
        pltpu.VMEM((CHUNK, 16), F32), pltpu.VMEM((CHUNK, 16), F32),
        pltpu.VMEM((CHUNK,), jnp.int32), pltpu.VMEM((CHUNK,), jnp.int32),
        pltpu.VMEM((CHUNK,), jnp.int32), pltpu.VMEM((CHUNK,), jnp.int32),
        pltpu.SemaphoreType.DMA, pltpu.SemaphoreType.DMA,
        pltpu.SemaphoreType.DMA, pltpu.SemaphoreType.DMA,
    ],
    compiler_params=pltpu.CompilerParams(needs_layout_passes=False,
                                         use_tc_tiling_on_sc=False),
)
def _sc_scatter_c(cpos_hbm, cneg_hbm, src_hbm, dst_hbm, cacc_out,
                  cacc_sp, zbuf2, cbuf0, cbuf1, cbuf20, cbuf21,
                  sidx0, sidx1, didx0, didx1,
                  sem_c0, sem_c1, sem_n0, sem_n1):
    cc = lax.axis_index("c")
    ss = lax.axis_index("s")
    z16 = jnp.zeros((16,), F32)
    cbuf = (cbuf0, cbuf1)
    cbuf2 = (cbuf20, cbuf21)
    sidx = (sidx0, sidx1)
    didx = (didx0, didx1)
    sem_c = (sem_c0, sem_c1)
    sem_n = (sem_n0, sem_n1)

    @pl.loop(0, _ZROWS)
    def _zero(i):
        zbuf2[i, :] = z16

    @pl.loop(0, ROW_SPAN // _ZROWS)
    def _zfill(j):
        pltpu.sync_copy(zbuf2, cacc_sp.at[pl.ds(ss * ROW_STRIDE + j * _ZROWS,
                                                _ZROWS)])
    plsc.subcore_barrier()

    def fire(slot, j):
        e0 = (cc * _CHUNKS_PER_SC + j) * CHUNK
        pltpu.sync_copy(src_hbm.at[pl.ds(e0, CHUNK)], sidx[slot])
        pltpu.sync_copy(dst_hbm.at[pl.ds(e0, CHUNK)], didx[slot])
        pltpu.async_copy(cpos_hbm.at[pl.ds(e0, CHUNK)], cbuf[slot], sem_c[slot])
        pltpu.async_copy(cneg_hbm.at[pl.ds(e0, CHUNK)], cbuf2[slot], sem_n[slot])

    @pl.when(ss < _CHUNKS_PER_SC)
    def _prologue():
        fire(0, ss)

    niter = (_CHUNKS_PER_SC + NS - 1) // NS

    @pl.loop(0, (niter + 1) // 2 * 2, step=2)
    def _outer(k):
        for b in range(2):
            p, np_ = b, 1 - b
            j = ss + (k + b) * NS

            @pl.when(j < _CHUNKS_PER_SC)
            def _(p=p, np_=np_, j=j):
                @pl.when(j + NS < _CHUNKS_PER_SC)
                def _():
                    fire(np_, j + NS)
                e0 = (cc * _CHUNKS_PER_SC + j) * CHUNK
                pltpu.make_async_copy(cpos_hbm.at[pl.ds(e0, CHUNK)],
                                      cbuf[p], sem_c[p]).wait()
                pltpu.sync_copy(cbuf[p], cacc_sp.at[didx[p]], add=True)
                pltpu.make_async_copy(cneg_hbm.at[pl.ds(e0, CHUNK)],
                                      cbuf2[p], sem_n[p]).wait()
                pltpu.sync_copy(cbuf2[p], cacc_sp.at[sidx[p]], add=True)

    plsc.subcore_barrier()
    r0 = ss * ROW_STRIDE
    pltpu.sync_copy(cacc_sp.at[pl.ds(r0, ROW_SPAN)],
                    cacc_out.at[cc, pl.ds(r0, ROW_SPAN)])


_BN = 2000
_BE = 8000


def kernel(h, x, bond_indices, W1, b1, W2, b2, Wn1, bn1, Wn2, bn2,
           Wc1, bc1, Wc2):
    h2 = h[0]
    x2 = x[0]
    src = bond_indices[:, 0].astype(jnp.int32)
    dst = bond_indices[:, 1].astype(jnp.int32)
    xx, xy, xz = x2[:, 0], x2[:, 1], x2[:, 2]
    w1a, w1b, w1c = W1[:H], W1[H:2 * H], W1[2 * H:2 * H + 1]
    b1r, b2r, bc1r = b1[None], b2[None], bc1[None]
    wn1a, wn1b = Wn1[:H], Wn1[H:2 * H]
    bn1r, bn2r = bn1[None], bn2[None]

    p_arr, q_arr = pl.pallas_call(
        _b0_body,
        grid=(N // _BN,),
        in_specs=[pl.BlockSpec((_BN, H), lambda i: (i, 0)),
                  pl.BlockSpec((H, H), lambda i: (0, 0)),
                  pl.BlockSpec((H, H), lambda i: (0, 0))],
        out_specs=[pl.BlockSpec((_BN, H), lambda i: (i, 0)),
                   pl.BlockSpec((_BN, H), lambda i: (i, 0))],
        out_shape=[jax.ShapeDtypeStruct((N, H), F32),
                   jax.ShapeDtypeStruct((N, H), F32)],
    )(h2, w1a, w1b)

    ps, qd, geo = _sc_gather(p_arr, q_arr, src, dst, xx, xy, xz)

    m_arr, cpos, cneg = pl.pallas_call(
        _edge_body,
        grid=(E // _BE,),
        in_specs=[pl.BlockSpec((_BE, H), lambda i: (i, 0)),
                  pl.BlockSpec((_BE, H), lambda i: (i, 0)),
                  pl.BlockSpec((_BE, 4), lambda i: (i, 0)),
                  pl.BlockSpec((1, H), lambda i: (0, 0)),
                  pl.BlockSpec((1, H), lambda i: (0, 0)),
                  pl.BlockSpec((H, H), lambda i: (0, 0)),
                  pl.BlockSpec((1, H), lambda i: (0, 0)),
                  pl.BlockSpec((H, H), lambda i: (0, 0)),
                  pl.BlockSpec((1, H), lambda i: (0, 0)),
                  pl.BlockSpec((H, 1), lambda i: (0, 0))],
        out_specs=[pl.BlockSpec((_BE, H), lambda i: (i, 0)),
                   pl.BlockSpec((_BE, 16), lambda i: (i, 0)),
                   pl.BlockSpec((_BE, 16), lambda i: (i, 0))],
        out_shape=[jax.ShapeDtypeStruct((E, H), F32),
                   jax.ShapeDtypeStruct((E, 16), F32),
                   jax.ShapeDtypeStruct((E, 16), F32)],
    )(ps, qd, geo, w1c, b1r, W2, b2r, Wc1, bc1r, Wc2)

    aggp = _sc_scatter_m(m_arr, src, dst)
    caccp = _sc_scatter_c(cpos, cneg, src, dst)

    h_out, x_out = pl.pallas_call(
        _node_body,
        grid=(N // _BN,),
        in_specs=[pl.BlockSpec((_BN, H), lambda i: (i, 0)),
                  pl.BlockSpec((_BN, 3), lambda i: (i, 0)),
                  pl.BlockSpec((NC, _BN, H), lambda i: (0, i, 0)),
                  pl.BlockSpec((NC, _BN, 16), lambda i: (0, i, 0)),
                  pl.BlockSpec((H, H), lambda i: (0, 0)),
                  pl.BlockSpec((H, H), lambda i: (0, 0)),
                  pl.BlockSpec((1, H), lambda i: (0, 0)),
                  pl.BlockSpec((H, H), lambda i: (0, 0)),
                  pl.BlockSpec((1, H), lambda i: (0, 0))],
        out_specs=[pl.BlockSpec((_BN, H), lambda i: (i, 0)),
                   pl.BlockSpec((_BN, 3), lambda i: (i, 0))],
        out_shape=[jax.ShapeDtypeStruct((N, H), F32),
                   jax.ShapeDtypeStruct((N, 3), F32)],
    )(h2, x2, aggp, caccp, wn1a, wn1b, bn1r, Wn2, bn2r)

    return h_out[None], x_out[None]


# speedup vs baseline: 2.3747x; 1.0011x over previous
import functools

import jax
import jax.numpy as jnp
from jax import lax
from jax.experimental import pallas as pl
from jax.experimental.pallas import tpu as pltpu
from jax.experimental.pallas import tpu_sc as plsc

N = 10000
H = 128
E = 320000
CHUNK = 128
NCHUNKS = E // CHUNK
NC, NS, NW = 2, 16, 32
ROW_STRIDE = 624
ROW_SPAN = 640
F32 = jnp.float32


def _b0_body(h_ref, wa_ref, wb_ref, p_ref, q_ref):
    hb = h_ref[...]
    p_ref[...] = jnp.dot(hb, wa_ref[...], preferred_element_type=F32)
    q_ref[...] = jnp.dot(hb, wb_ref[...], preferred_element_type=F32)


def _edge_body(ps_ref, qd_ref, geo_ref, w1c_ref, b1_ref, w2_ref, b2_ref,
               wc1_ref, bc1_ref, wc2_ref, m_ref, cpos_ref, cneg_ref):
    g = geo_ref[...]
    dist = jnp.sqrt(g[:, 3:4])
    pre = ps_ref[...] + qd_ref[...] + dist * w1c_ref[...] + b1_ref[...]
    t1 = jax.nn.silu(pre)
    mm = jax.nn.silu(jnp.dot(t1, w2_ref[...], preferred_element_type=F32)
                     + b2_ref[...])
    m_ref[...] = mm
    t2 = jax.nn.silu(jnp.dot(mm, wc1_ref[...], preferred_element_type=F32)
                     + bc1_ref[...])
    cw = jnp.dot(t2, wc2_ref[...], preferred_element_type=F32)
    s = cw / (dist + 1e-8)
    cvec = g[:, 0:3] * s
    ones = jnp.ones_like(cw)
    zeros = jnp.zeros(cvec.shape[:1] + (12,), F32)
    cpos_ref[...] = jnp.concatenate([cvec, ones, zeros], axis=1)
    cneg_ref[...] = jnp.concatenate([-cvec, ones, zeros], axis=1)


def _node_body(h_ref, x_ref, agg_ref, cacc_ref, wn1a_ref, wn1b_ref, bn1_ref,
               wn2_ref, bn2_ref, ho_ref, xo_ref):
    a3 = agg_ref[...]
    a = a3[0] + a3[1]
    hb = h_ref[...]
    t = jax.nn.silu(jnp.dot(hb, wn1a_ref[...], preferred_element_type=F32)
                    + jnp.dot(a, wn1b_ref[...], preferred_element_type=F32)
                    + bn1_ref[...])
    ho_ref[...] = hb + jnp.dot(t, wn2_ref[...], preferred_element_type=F32) \
        + bn2_ref[...]
    c3 = cacc_ref[...]
    c = c3[0] + c3[1]
    cnt = jnp.maximum(c[:, 3:4], 1.0)
    xo_ref[...] = x_ref[...] + c[:, 0:3] / cnt


_MESH = plsc.VectorSubcoreMesh(core_axis_name="c", subcore_axis_name="s",
                               num_cores=NC, num_subcores=NS)


@functools.partial(
    pl.kernel,
    out_type=(jax.ShapeDtypeStruct((E, H), F32),
              jax.ShapeDtypeStruct((E, H), F32),
              jax.ShapeDtypeStruct((E, 4), F32)),
    mesh=_MESH,
    scratch_types=[
        pltpu.VMEM((N,), F32), pltpu.VMEM((N,), F32), pltpu.VMEM((N,), F32),
        pltpu.VMEM((CHUNK,), jnp.int32), pltpu.VMEM((CHUNK,), jnp.int32),
        pltpu.VMEM((CHUNK,), jnp.int32), pltpu.VMEM((CHUNK,), jnp.int32),
        pltpu.VMEM((CHUNK, H), F32), pltpu.VMEM((CHUNK, H), F32),
        pltpu.VMEM((CHUNK, H), F32), pltpu.VMEM((CHUNK, H), F32),
        pltpu.VMEM((CHUNK, 4), F32),
        pltpu.SemaphoreType.DMA, pltpu.SemaphoreType.DMA,
        pltpu.SemaphoreType.DMA, pltpu.SemaphoreType.DMA,
        pltpu.SemaphoreType.DMA, pltpu.SemaphoreType.DMA,
        pltpu.SemaphoreType.DMA, pltpu.SemaphoreType.DMA,
    ],
    compiler_params=pltpu.CompilerParams(needs_layout_passes=False),
)
def _sc_gather(p_hbm, q_hbm, src_hbm, dst_hbm, xx_hbm, xy_hbm, xz_hbm,
               ps_hbm, qd_hbm, geo_hbm,
               xx_v, xy_v, xz_v, sidx0, sidx1, didx0, didx1,
               pbuf0, pbuf1, qbuf0, qbuf1, gbuf,
               sem_p0, sem_p1, sem_q0, sem_q1,
               sem_wp0, sem_wp1, sem_wq0, sem_wq1):
    cc = lax.axis_index("c")
    ss = lax.axis_index("s")
    wid = ss * NC + cc
    pltpu.sync_copy(xx_hbm, xx_v)
    pltpu.sync_copy(xy_hbm, xy_v)
    pltpu.sync_copy(xz_hbm, xz_v)
    iota = lax.iota(jnp.int32, 16)
    sidx = (sidx0, sidx1)
    didx = (didx0, didx1)
    pbuf = (pbuf0, pbuf1)
    qbuf = (qbuf0, qbuf1)
    sem_p = (sem_p0, sem_p1)
    sem_q = (sem_q0, sem_q1)
    sem_wp = (sem_wp0, sem_wp1)
    sem_wq = (sem_wq0, sem_wq1)
    niter = (NCHUNKS + NW - 1) // NW

    def drain_writes(slot):
        pltpu.make_async_copy(pbuf[slot], ps_hbm.at[pl.ds(0, CHUNK)],
                              sem_wp[slot]).wait()
        pltpu.make_async_copy(qbuf[slot], qd_hbm.at[pl.ds(0, CHUNK)],
                              sem_wq[slot]).wait()

    def fire(slot, n):
        e0 = n * CHUNK
        pltpu.sync_copy(src_hbm.at[pl.ds(e0, CHUNK)], sidx[slot])
        pltpu.sync_copy(dst_hbm.at[pl.ds(e0, CHUNK)], didx[slot])
        pltpu.async_copy(p_hbm.at[sidx[slot]], pbuf[slot], sem_p[slot])
        pltpu.async_copy(q_hbm.at[didx[slot]], qbuf[slot], sem_q[slot])

    @pl.when(wid < NCHUNKS)
    def _prologue():
        fire(0, wid)

    @pl.loop(0, (niter + 1) // 2 * 2, step=2)
    def _outer(k):
        for b in range(2):
            p, np_ = b, 1 - b
            n = wid + (k + b) * NW

            @pl.when(n < NCHUNKS)
            def _(p=p, np_=np_, n=n):
                e0 = n * CHUNK
                n_next = n + NW

                @pl.when(n_next < NCHUNKS)
                def _():
                    @pl.when(n_next >= wid + 2 * NW)
                    def _():
                        drain_writes(np_)
                    fire(np_, n_next)

                for sub in range(CHUNK // 16):
                    sv = sidx[p][pl.ds(sub * 16, 16)]
                    dv = didx[p][pl.ds(sub * 16, 16)]
                    dxv = plsc.load_gather(xx_v, [dv]) - plsc.load_gather(xx_v, [sv])
                    dyv = plsc.load_gather(xy_v, [dv]) - plsc.load_gather(xy_v, [sv])
                    dzv = plsc.load_gather(xz_v, [dv]) - plsc.load_gather(xz_v, [sv])
                    sqv = dxv * dxv + dyv * dyv + dzv * dzv
                    rows = iota + (sub * 16)
                    for comp, val in ((0, dxv), (1, dyv), (2, dzv), (3, sqv)):
                        plsc.store_scatter(
                            gbuf, [rows, jnp.full((16,), comp, jnp.int32)], val)
                pltpu.sync_copy(gbuf, geo_hbm.at[pl.ds(e0, CHUNK)])
                pltpu.make_async_copy(p_hbm.at[sidx[p]], pbuf[p], sem_p[p]).wait()
                pltpu.async_copy(pbuf[p], ps_hbm.at[pl.ds(e0, CHUNK)], sem_wp[p])
                pltpu.make_async_copy(q_hbm.at[didx[p]], qbuf[p], sem_q[p]).wait()
                pltpu.async_copy(qbuf[p], qd_hbm.at[pl.ds(e0, CHUNK)], sem_wq[p])

    drain_writes(0)
    drain_writes(1)


_CHUNKS_PER_SC = NCHUNKS // NC
_ZROWS = 16


@functools.partial(
    pl.kernel,
    out_type=jax.ShapeDtypeStruct((NC, N, H), F32),
    mesh=_MESH,
    scratch_types=[
        pltpu.VMEM_SHARED((N, H), F32),
        pltpu.VMEM((_ZROWS, H), F32),
        pltpu.VMEM((CHUNK, H), F32), pltpu.VMEM((CHUNK, H), F32),
        pltpu.VMEM((CHUNK,), jnp.int32), pltpu.VMEM((CHUNK,), jnp.int32),
        pltpu.VMEM((CHUNK,), jnp.int32), pltpu.VMEM((CHUNK,), jnp.int32),
        pltpu.SemaphoreType.DMA, pltpu.SemaphoreType.DMA,
    ],
    compiler_params=pltpu.CompilerParams(needs_layout_passes=False),
)
def _sc_scatter_m(m_hbm, src_hbm, dst_hbm, agg_out,
                  agg_sp, zbuf, mbuf0, mbuf1, sidx0, sidx1, didx0, didx1,
                  sem_m0, sem_m1):
    cc = lax.axis_index("c")
    ss = lax.axis_index("s")
    z16 = jnp.zeros((16,), F32)
    mbuf = (mbuf0, mbuf1)
    sidx = (sidx0, sidx1)
    didx = (didx0, didx1)
    sem_m = (sem_m0, sem_m1)

    @pl.loop(0, _ZROWS)
    def _zero(i):
        for j in range(H // 16):
            zbuf[i, pl.ds(j * 16, 16)] = z16

    @pl.loop(0, ROW_SPAN // _ZROWS)
    def _zfill(j):
        pltpu.sync_copy(zbuf, agg_sp.at[pl.ds(ss * ROW_STRIDE + j * _ZROWS,
                                               _ZROWS)])
    plsc.subcore_barrier()

    def fire(slot, j):
        e0 = (cc * _CHUNKS_PER_SC + j) * CHUNK
        pltpu.sync_copy(src_hbm.at[pl.ds(e0, CHUNK)], sidx[slot])
        pltpu.sync_copy(dst_hbm.at[pl.ds(e0, CHUNK)], didx[slot])
        pltpu.async_copy(m_hbm.at[pl.ds(e0, CHUNK)], mbuf[slot], sem_m[slot])

    @pl.when(ss < _CHUNKS_PER_SC)
    def _prologue():
        fire(0, ss)

    niter = (_CHUNKS_PER_SC + NS - 1) // NS

    @pl.loop(0, (niter + 1) // 2 * 2, step=2)
    def _outer(k):
        for b in range(2):
            p, np_ = b, 1 - b
            j = ss + (k + b) * NS

            @pl.when(j < _CHUNKS_PER_SC)
            def _(p=p, np_=np_, j=j):
                @pl.when(j + NS < _CHUNKS_PER_SC)
                def _():
                    fire(np_, j + NS)
                e0 = (cc * _CHUNKS_PER_SC + j) * CHUNK
                pltpu.make_async_copy(m_hbm.at[pl.ds(e0, CHUNK)],
                                      mbuf[p], sem_m[p]).wait()
                pltpu.sync_copy(mbuf[p], agg_sp.at[didx[p]], add=True)
                pltpu.sync_copy(mbuf[p], agg_sp.at[sidx[p]], add=True)

    plsc.subcore_barrier()
    r0 = ss * ROW_STRIDE
    pltpu.sync_copy(agg_sp.at[pl.ds(r0, ROW_SPAN)],
                    agg_out.at[cc, pl.ds(r0, ROW_SPAN)])


@functools.partial(
    pl.kernel,
    out_type=jax.ShapeDtypeStruct((NC, N, 16), F32),
    mesh=_MESH,
    scratch_types=[
        pltpu.VMEM_SHARED((N, 16), F32),
        pltpu.VMEM((_ZROWS, 16), F32),
        pltpu.VMEM((CHUNK, 16), F32), pltpu.VMEM((CHUNK, 16), F32),
        pltpu.VMEM((CHUNK, 16), F32), pltpu.VMEM((CHUNK, 16), F32),
        pltpu.VMEM((CHUNK,), jnp.int32), pltpu.VMEM((CHUNK,), jnp.int32),
        pltpu.VMEM((CHUNK,), jnp.int32), pltpu.VMEM((CHUNK,), jnp.int32),
        pltpu.SemaphoreType.DMA, pltpu.SemaphoreType.DMA,
        pltpu.SemaphoreType.DMA, pltpu.SemaphoreType.DMA,
    ],
    compiler_params=pltpu.CompilerParams(needs_layout_passes=False,
                                         use_tc_tiling_on_sc=False),
)
def _sc_scatter_c(cpos_hbm, cneg_hbm, src_hbm, dst_hbm, cacc_out,
                  cacc_sp, zbuf2, cbuf0, cbuf1, cbuf20, cbuf21,
                  sidx0, sidx1, didx0, didx1,
                  sem_c0, sem_c1, sem_n0, sem_n1):
    cc = lax.axis_index("c")
    ss = lax.axis_index("s")
    z16 = jnp.zeros((16,), F32)
    cbuf = (cbuf0, cbuf1)
    cbuf2 = (cbuf20, cbuf21)
    sidx = (sidx0, sidx1)
    didx = (didx0, didx1)
    sem_c = (sem_c0, sem_c1)
    sem_n = (sem_n0, sem_n1)

    @pl.loop(0, _ZROWS)
    def _zero(i):
        zbuf2[i, :] = z16

    @pl.loop(0, ROW_SPAN // _ZROWS)
    def _zfill(j):
        pltpu.sync_copy(zbuf2, cacc_sp.at[pl.ds(ss * ROW_STRIDE + j * _ZROWS,
                                                _ZROWS)])
    plsc.subcore_barrier()

    def fire(slot, j):
        e0 = (cc * _CHUNKS_PER_SC + j) * CHUNK
        pltpu.sync_copy(src_hbm.at[pl.ds(e0, CHUNK)], sidx[slot])
        pltpu.sync_copy(dst_hbm.at[pl.ds(e0, CHUNK)], didx[slot])
        pltpu.async_copy(cpos_hbm.at[pl.ds(e0, CHUNK)], cbuf[slot], sem_c[slot])
        pltpu.async_copy(cneg_hbm.at[pl.ds(e0, CHUNK)], cbuf2[slot], sem_n[slot])

    @pl.when(ss < _CHUNKS_PER_SC)
    def _prologue():
        fire(0, ss)

    niter = (_CHUNKS_PER_SC + NS - 1) // NS

    @pl.loop(0, (niter + 1) // 2 * 2, step=2)
    def _outer(k):
        for b in range(2):
            p, np_ = b, 1 - b
            j = ss + (k + b) * NS

            @pl.when(j < _CHUNKS_PER_SC)
            def _(p=p, np_=np_, j=j):
                @pl.when(j + NS < _CHUNKS_PER_SC)
                def _():
                    fire(np_, j + NS)
                e0 = (cc * _CHUNKS_PER_SC + j) * CHUNK
                pltpu.make_async_copy(cpos_hbm.at[pl.ds(e0, CHUNK)],
                                      cbuf[p], sem_c[p]).wait()
                pltpu.sync_copy(cbuf[p], cacc_sp.at[didx[p]], add=True)
                pltpu.make_async_copy(cneg_hbm.at[pl.ds(e0, CHUNK)],
                                      cbuf2[p], sem_n[p]).wait()
                pltpu.sync_copy(cbuf2[p], cacc_sp.at[sidx[p]], add=True)

    plsc.subcore_barrier()
    r0 = ss * ROW_STRIDE
    pltpu.sync_copy(cacc_sp.at[pl.ds(r0, ROW_SPAN)],
                    cacc_out.at[cc, pl.ds(r0, ROW_SPAN)])


_BN = 2000
_BE = 8000


def kernel(h, x, bond_indices, W1, b1, W2, b2, Wn1, bn1, Wn2, bn2,
           Wc1, bc1, Wc2):
    h2 = h[0]
    x2 = x[0]
    src = bond_indices[:, 0].astype(jnp.int32)
    dst = bond_indices[:, 1].astype(jnp.int32)
    xx, xy, xz = x2[:, 0], x2[:, 1], x2[:, 2]
    w1a, w1b, w1c = W1[:H], W1[H:2 * H], W1[2 * H:2 * H + 1]
    b1r, b2r, bc1r = b1[None], b2[None], bc1[None]
    wn1a, wn1b = Wn1[:H], Wn1[H:2 * H]
    bn1r, bn2r = bn1[None], bn2[None]

    p_arr, q_arr = pl.pallas_call(
        _b0_body,
        grid=(N // _BN,),
        in_specs=[pl.BlockSpec((_BN, H), lambda i: (i, 0)),
                  pl.BlockSpec((H, H), lambda i: (0, 0)),
                  pl.BlockSpec((H, H), lambda i: (0, 0))],
        out_specs=[pl.BlockSpec((_BN, H), lambda i: (i, 0)),
                   pl.BlockSpec((_BN, H), lambda i: (i, 0))],
        out_shape=[jax.ShapeDtypeStruct((N, H), F32),
                   jax.ShapeDtypeStruct((N, H), F32)],
    )(h2, w1a, w1b)

    ps, qd, geo = _sc_gather(p_arr, q_arr, src, dst, xx, xy, xz)

    m_arr, cpos, cneg = pl.pallas_call(
        _edge_body,
        grid=(E // _BE,),
        in_specs=[pl.BlockSpec((_BE, H), lambda i: (i, 0)),
                  pl.BlockSpec((_BE, H), lambda i: (i, 0)),
                  pl.BlockSpec((_BE, 4), lambda i: (i, 0)),
                  pl.BlockSpec((1, H), lambda i: (0, 0)),
                  pl.BlockSpec((1, H), lambda i: (0, 0)),
                  pl.BlockSpec((H, H), lambda i: (0, 0)),
                  pl.BlockSpec((1, H), lambda i: (0, 0)),
                  pl.BlockSpec((H, H), lambda i: (0, 0)),
                  pl.BlockSpec((1, H), lambda i: (0, 0)),
                  pl.BlockSpec((H, 1), lambda i: (0, 0))],
        out_specs=[pl.BlockSpec((_BE, H), lambda i: (i, 0)),
                   pl.BlockSpec((_BE, 16), lambda i: (i, 0)),
                   pl.BlockSpec((_BE, 16), lambda i: (i, 0))],
        out_shape=[jax.ShapeDtypeStruct((E, H), F32),
                   jax.ShapeDtypeStruct((E, 16), F32),
                   jax.ShapeDtypeStruct((E, 16), F32)],
    )(ps, qd, geo, w1c, b1r, W2, b2r, Wc1, bc1r, Wc2)

    aggp = _sc_scatter_m(m_arr, src, dst)
    caccp = _sc_scatter_c(cpos, cneg, src, dst)

    h_out, x_out = pl.pallas_call(
        _node_body,
        grid=(N // _BN,),
        in_specs=[pl.BlockSpec((_BN, H), lambda i: (i, 0)),
                  pl.BlockSpec((_BN, 3), lambda i: (i, 0)),
                  pl.BlockSpec((NC, _BN, H), lambda i: (0, i, 0)),
                  pl.BlockSpec((NC, _BN, 16), lambda i: (0, i, 0)),
                  pl.BlockSpec((H, H), lambda i: (0, 0)),
                  pl.BlockSpec((H, H), lambda i: (0, 0)),
                  pl.BlockSpec((1, H), lambda i: (0, 0)),
                  pl.BlockSpec((H, H), lambda i: (0, 0)),
                  pl.BlockSpec((1, H), lambda i: (0, 0))],
        out_specs=[pl.BlockSpec((_BN, H), lambda i: (i, 0)),
                   pl.BlockSpec((_BN, 3), lambda i: (i, 0))],
        out_shape=[jax.ShapeDtypeStruct((N, H), F32),
                   jax.ShapeDtypeStruct((N, 3), F32)],
    )(h2, x2, aggp, caccp, wn1a, wn1b, bn1r, Wn2, bn2r)

    return h_out[None], x_out[None]
